# baseline (speedup 1.0000x reference)
import functools

import jax
import jax.numpy as jnp
from jax import lax
from jax.experimental import pallas as pl
from jax.experimental.pallas import tpu as pltpu
from jax.experimental.pallas import tpu_sc as plsc

NC = 2
NS = 16
NW = NC * NS

RBLK = 16
L = 16
PADW = 128


def _make_gather(n_rows: int, n_cols: int, dim: int):
  assert n_rows % (NW * RBLK) == 0
  rows_per_w = n_rows // NW
  n_blk = rows_per_w // RBLK
  assert n_blk % 2 == 0 and n_blk >= 4
  n_chunk = 2
  chunk_rows = rows_per_w // n_chunk

  mesh = plsc.VectorSubcoreMesh(core_axis_name="c", subcore_axis_name="s")

  @functools.partial(
      pl.kernel,
      mesh=mesh,
      compiler_params=pltpu.CompilerParams(use_tc_tiling_on_sc=False,
                                           needs_layout_passes=False),
      out_type=jax.ShapeDtypeStruct((n_rows, n_cols, dim), jnp.float32),
      scratch_types=[
          pltpu.VMEM((chunk_rows, PADW), jnp.int32),
          pltpu.VMEM((rows_per_w, n_cols), jnp.int32),
          pltpu.VMEM((2, RBLK, n_cols, dim), jnp.float32),
          pltpu.SemaphoreType.DMA,
          pltpu.SemaphoreType.DMA,
          pltpu.SemaphoreType.DMA,
          pltpu.SemaphoreType.DMA,
      ],
  )
  def gather_kernel(idx_hbm, table_hbm, out_hbm, slab_v, idx_v, rows_v,
                    gsem0, gsem1, osem0, osem1):
    wid = lax.axis_index("s") * NC + lax.axis_index("c")
    row0 = wid * rows_per_w
    gsem = (gsem0, gsem1)
    osem = (osem0, osem1)
    lanes = lax.iota(jnp.int32, L)

    for c in range(n_chunk):
      pltpu.sync_copy(
          idx_hbm.at[pl.ds(row0 + c * chunk_rows, chunk_rows), :], slab_v)

      @pl.loop(0, chunk_rows)
      def _row(r):
        dst = r + c * chunk_rows
        for lo in range(0, n_cols, L):
          width = min(L, n_cols - lo)
          v = slab_v[r, pl.ds(lo, L)]
          if width == L:
            idx_v[dst, pl.ds(lo, L)] = v
          else:
            rows = jnp.full((L,), 0, jnp.int32) + dst
            cols = jnp.minimum(lanes + lo, n_cols - 1)
            plsc.store_scatter(idx_v, [rows, cols], v, mask=lanes < width)

    def fire_gather(j, b):
      for r in range(RBLK):
        pltpu.async_copy(
            table_hbm.at[idx_v.at[j * RBLK + r]],
            rows_v.at[b].at[r], gsem[b])

    def drain_gather(b):
      pltpu.make_async_copy(
          out_hbm.at[pl.ds(row0, RBLK), :, :], rows_v.at[b], gsem[b]).wait()

    def fire_writeback(j, b):
      pltpu.async_copy(
          rows_v.at[b], out_hbm.at[pl.ds(row0 + j * RBLK, RBLK), :, :],
          osem[b])

    def drain_writeback(b):
      pltpu.make_async_copy(
          rows_v.at[b], out_hbm.at[pl.ds(row0, RBLK), :, :], osem[b]).wait()

    fire_gather(0, 0)
    fire_gather(1, 1)
    drain_gather(0)
    fire_writeback(0, 0)

    @pl.loop(2, n_blk, step=2)
    def _steady(i):
      for b in range(2):
        j = i + b
        drain_writeback(b)
        fire_gather(j, b)
        drain_gather(1 - b)
        fire_writeback(j - 1, 1 - b)

    drain_gather(1)
    fire_writeback(n_blk - 1, 1)
    drain_writeback(0)
    drain_writeback(1)

  return gather_kernel


def kernel(indices, items_emb):
  n_rows, n_cols = indices.shape
  m, dim = items_emb.shape
  idx = jnp.pad(indices.astype(jnp.int32), ((0, 0), (0, PADW - n_cols)))
  return _make_gather(n_rows, n_cols, dim)(idx, items_emb)


# speedup vs baseline: 1.0073x; 1.0073x over previous
import functools

import jax
import jax.numpy as jnp
from jax import lax
from jax.experimental import pallas as pl
from jax.experimental.pallas import tpu as pltpu
from jax.experimental.pallas import tpu_sc as plsc

NC = 2
NS = 16
NW = NC * NS

RBLK = 16


def _make_gather(n_rows: int, n_cols: int, dim: int):
  assert n_rows % (NW * RBLK) == 0
  rows_per_w = n_rows // NW
  n_blk = rows_per_w // RBLK
  assert n_blk % 2 == 0 and n_blk >= 4

  mesh = plsc.VectorSubcoreMesh(core_axis_name="c", subcore_axis_name="s")

  @functools.partial(
      pl.kernel,
      mesh=mesh,
      compiler_params=pltpu.CompilerParams(use_tc_tiling_on_sc=False),
      out_type=jax.ShapeDtypeStruct((n_rows, n_cols, dim), jnp.float32),
      scratch_types=[
          pltpu.VMEM((rows_per_w, n_cols), jnp.int32),
          pltpu.VMEM((2, RBLK, n_cols, dim), jnp.float32),
          pltpu.SemaphoreType.DMA,
          pltpu.SemaphoreType.DMA,
          pltpu.SemaphoreType.DMA,
          pltpu.SemaphoreType.DMA,
      ],
  )
  def gather_kernel(idx_hbm, table_hbm, out_hbm, idx_v, rows_v,
                    gsem0, gsem1, osem0, osem1):
    wid = lax.axis_index("s") * NC + lax.axis_index("c")
    row0 = wid * rows_per_w
    gsem = (gsem0, gsem1)
    osem = (osem0, osem1)

    pltpu.sync_copy(idx_hbm.at[pl.ds(row0, rows_per_w), :], idx_v)

    def fire_gather(j, b):
      for r in range(RBLK):
        pltpu.async_copy(
            table_hbm.at[idx_v.at[j * RBLK + r]],
            rows_v.at[b].at[r], gsem[b])

    def drain_gather(b):
      pltpu.make_async_copy(
          out_hbm.at[pl.ds(row0, RBLK), :, :], rows_v.at[b], gsem[b]).wait()

    def fire_writeback(j, b):
      pltpu.async_copy(
          rows_v.at[b], out_hbm.at[pl.ds(row0 + j * RBLK, RBLK), :, :],
          osem[b])

    def drain_writeback(b):
      pltpu.make_async_copy(
          rows_v.at[b], out_hbm.at[pl.ds(row0, RBLK), :, :], osem[b]).wait()

    fire_gather(0, 0)
    fire_gather(1, 1)
    drain_gather(0)
    fire_writeback(0, 0)

    @pl.loop(2, n_blk, step=2)
    def _steady(i):
      for b in range(2):
        j = i + b
        drain_writeback(b)
        fire_gather(j, b)
        drain_gather(1 - b)
        fire_writeback(j - 1, 1 - b)

    drain_gather(1)
    fire_writeback(n_blk - 1, 1)
    drain_writeback(0)
    drain_writeback(1)

  return gather_kernel


def kernel(indices, items_emb):
  n_rows, n_cols = indices.shape
  m, dim = items_emb.shape
  return _make_gather(n_rows, n_cols, dim)(
      indices.astype(jnp.int32), items_emb)
